# baseline (speedup 1.0000x reference)
import jax
import jax.numpy as jnp
from jax import lax
from jax.experimental import pallas as pl
from jax.experimental.pallas import tpu as pltpu
from jax.experimental.pallas import tpu_sc as plsc

BS, SEQ, D, PLEN, SESSIONS = 256, 196, 768, 10, 10
OUT_SEQ = PLEN + SEQ
NC, NS = 2, 16
NW = NC * NS
SPW = BS // NW
NCHUNK = 7
CHROWS = SEQ // NCHUNK
TOT = SPW * NCHUNK


def _sc_body(x_hbm, pool_hbm, tid_hbm, out_hbm, idx_v, pv, bufs,
             sem_g, sem_in, sem_out, sem_p):
    wid = lax.axis_index("s") * NC + lax.axis_index("c")
    base = pl.multiple_of(wid * SPW, SPW)

    pltpu.sync_copy(tid_hbm.at[pl.ds(base, SPW)], idx_v)
    gather = pltpu.make_async_copy(pool_hbm.at[idx_v], pv, sem_g)
    gather.start()

    def in_copy(c, b):
        j, p = c // NCHUNK, c % NCHUNK
        return pltpu.make_async_copy(
            x_hbm.at[base + j, pl.ds(p * CHROWS, CHROWS)],
            bufs.at[b], sem_in.at[b])

    def out_copy(c, b):
        j, p = c // NCHUNK, c % NCHUNK
        return pltpu.make_async_copy(
            bufs.at[b],
            out_hbm.at[base + j, pl.ds(PLEN + p * CHROWS, CHROWS)],
            sem_out.at[b])

    in_copy(0, 0).start()
    in_copy(0, 0).wait()
    out_copy(0, 0).start()
    in_copy(1, 1).start()

    def group(g, carry):
        c1 = 2 * g + 1
        in_copy(c1, 1).wait()
        out_copy(c1, 1).start()
        out_copy(c1 - 1, 0).wait()
        in_copy(c1 + 1, 0).start()
        c2 = 2 * g + 2
        in_copy(c2, 0).wait()
        out_copy(c2, 0).start()
        out_copy(c2 - 1, 1).wait()
        in_copy(c2 + 1, 1).start()
        return carry

    lax.fori_loop(0, (TOT - 2) // 2, group, 0)
    c = TOT - 1
    in_copy(c, 1).wait()
    out_copy(c, 1).start()
    out_copy(c - 1, 0).wait()
    out_copy(c, 1).wait()

    gather.wait()
    p_writes = [
        pltpu.make_async_copy(
            pv.at[j], out_hbm.at[base + j, pl.ds(0, PLEN)], sem_p)
        for j in range(SPW)
    ]
    for w in p_writes:
        w.start()
    for w in p_writes:
        w.wait()


def kernel(x, prompt_pool, task_id):
    mesh = plsc.VectorSubcoreMesh(core_axis_name="c", subcore_axis_name="s")
    run = pl.kernel(
        _sc_body,
        out_type=jax.ShapeDtypeStruct((BS, OUT_SEQ, D), jnp.float32),
        mesh=mesh,
        compiler_params=pltpu.CompilerParams(use_tc_tiling_on_sc=False),
        scratch_types=[
            pltpu.VMEM((SPW,), jnp.int32),
            pltpu.VMEM((SPW, PLEN, D), jnp.float32),
            pltpu.VMEM((2, CHROWS, D), jnp.float32),
            pltpu.SemaphoreType.DMA,
            pltpu.SemaphoreType.DMA((2,)),
            pltpu.SemaphoreType.DMA((2,)),
            pltpu.SemaphoreType.DMA,
        ],
    )
    return run(x, prompt_pool, task_id.astype(jnp.int32))


# speedup vs baseline: 1.7944x; 1.7944x over previous
import jax
import jax.numpy as jnp
from jax.experimental import pallas as pl
from jax.experimental.pallas import tpu as pltpu

BS, SEQ, D, PLEN, SESSIONS = 256, 196, 768, 10, 10
OUT_SEQ = PLEN + SEQ
SPB = 4
NSTEP = BS // SPB


def _body(tid_ref, x_ref, pool_ref, out_ref):
    i = pl.program_id(0)
    for j in range(SPB):
        t = tid_ref[i * SPB + j]
        out_ref[j, :PLEN, :] = pool_ref[t]
        out_ref[j, PLEN:, :] = x_ref[j]


def kernel(x, prompt_pool, task_id):
    grid_spec = pltpu.PrefetchScalarGridSpec(
        num_scalar_prefetch=1,
        grid=(NSTEP,),
        in_specs=[
            pl.BlockSpec((SPB, SEQ, D), lambda i, tid: (i, 0, 0)),
            pl.BlockSpec((SESSIONS, PLEN, D), lambda i, tid: (0, 0, 0)),
        ],
        out_specs=pl.BlockSpec((SPB, OUT_SEQ, D), lambda i, tid: (i, 0, 0)),
    )
    return pl.pallas_call(
        _body,
        grid_spec=grid_spec,
        out_shape=jax.ShapeDtypeStruct((BS, OUT_SEQ, D), x.dtype),
    )(task_id.astype(jnp.int32), x, prompt_pool)
